# baseline (speedup 1.0000x reference)
import functools

import jax
import jax.numpy as jnp
from jax import lax
from jax.experimental import pallas as pl
from jax.experimental.pallas import tpu as pltpu
from jax.experimental.pallas import tpu_sc as plsc

TAU = 10.0
TOPK = 8
K = 64
D = 2048
TOKEN_BLOCK = 1024

_SC_INFO = plsc.get_sparse_core_info()
_NWORKERS = _SC_INFO.num_cores * _SC_INFO.num_subcores
_LANES = _SC_INFO.num_lanes


def _logits_block(z_ref, cb_ref, lt_ref):
    z = z_ref[...]
    cb = cb_ref[...]
    raw = lax.dot_general(
        cb, z, (((1,), (1,)), ((), ())), preferred_element_type=jnp.float32)
    z2 = lax.dot_general(
        jnp.ones((1, z.shape[1]), jnp.float32), z * z,
        (((1,), (1,)), ((), ())), preferred_element_type=jnp.float32)
    rz = TAU / jnp.maximum(jnp.sqrt(z2), 1e-12)
    c2 = lax.dot_general(
        cb * cb, jnp.ones((z.shape[1], 8), jnp.float32),
        (((1,), (0,)), ((), ())), preferred_element_type=jnp.float32)[:, :1]
    rc = 1.0 / jnp.maximum(jnp.sqrt(c2), 1e-12)
    lt_ref[...] = raw * rz * rc


def _route_sc(bn):
    tpw = bn // _NWORKERS
    groups = tpw // _LANES
    mesh = plsc.VectorSubcoreMesh(core_axis_name="c", subcore_axis_name="s")

    @functools.partial(
        pl.kernel,
        mesh=mesh,
        out_type=jax.ShapeDtypeStruct((bn,), jnp.float32),
        scratch_types=[
            pltpu.VMEM((K, tpw), jnp.float32),
            pltpu.VMEM((tpw,), jnp.float32),
        ],
    )
    def route(lt_hbm, thr_hbm, lv, tv):
        wid = lax.axis_index("s") * _SC_INFO.num_cores + lax.axis_index("c")
        base = wid * tpw
        pltpu.sync_copy(lt_hbm.at[:, pl.ds(base, tpw)], lv)

        def body(g, carry):
            neg = jnp.full((_LANES,), -jnp.inf, jnp.float32)
            s = [neg] * TOPK
            col = g * _LANES
            for k in range(K):
                x = lv[k, pl.ds(col, _LANES)]
                for i in range(TOPK):
                    hi = jnp.maximum(s[i], x)
                    x = jnp.minimum(s[i], x)
                    s[i] = hi
            tv[pl.ds(col, _LANES)] = s[TOPK - 1]
            return carry

        lax.fori_loop(0, groups, body, 0)
        pltpu.sync_copy(tv, thr_hbm.at[pl.ds(base, tpw)])

    return route


def _combine_block(lt1_ref, lt2_ref, thr1_ref, thr2_ref, t_ref, e_ref, o_ref):
    first = pl.program_id(0) < (pl.num_programs(0) // 2)
    lt = jnp.where(first, lt1_ref[...], lt2_ref[...])
    thr = jnp.where(first, thr1_ref[...], thr2_ref[...])
    e = jnp.where(lt >= thr, jnp.exp(lt - thr), 0.0)
    s = jnp.sum(e, axis=0, keepdims=True)
    w = e * (1.0 / s)
    g = lax.dot_general(
        w, e_ref[...], (((0,), (0,)), ((), ())), preferred_element_type=jnp.float32)
    o_ref[...] = t_ref[...] * (1.0 + g)


def _logits_stage(zf, codebook, d, c, cn):
    nb = cn // TOKEN_BLOCK
    return pl.pallas_call(
        _logits_block,
        grid=(nb,),
        in_specs=[
            pl.BlockSpec((TOKEN_BLOCK, d), lambda i: (i + c * nb, 0)),
            pl.BlockSpec((K, d), lambda i: (0, 0)),
        ],
        out_specs=pl.BlockSpec((K, TOKEN_BLOCK), lambda i: (0, i)),
        out_shape=jax.ShapeDtypeStruct((K, cn), jnp.float32),
        compiler_params=pltpu.CompilerParams(
            dimension_semantics=("arbitrary",),
        ),
    )(zf, codebook)


@functools.partial(jax.jit, static_argnames=())
def kernel(z, target, codebook, E):
    b, n, d = z.shape
    bn = b * n
    cn = bn // 2
    nb = cn // TOKEN_BLOCK
    zf = z.reshape(bn, d)
    tf = target.reshape(bn, d)

    route = _route_sc(cn)
    lt1 = _logits_stage(zf, codebook, d, 0, cn)
    thr1 = route(lt1)
    lt2 = _logits_stage(zf, codebook, d, 1, cn)
    thr2 = route(lt2)

    out = pl.pallas_call(
        _combine_block,
        grid=(2 * nb,),
        in_specs=[
            pl.BlockSpec((K, TOKEN_BLOCK), lambda i: (0, jnp.minimum(i, nb - 1))),
            pl.BlockSpec((K, TOKEN_BLOCK), lambda i: (0, jnp.maximum(i - nb, 0))),
            pl.BlockSpec((1, TOKEN_BLOCK), lambda i: (0, jnp.minimum(i, nb - 1))),
            pl.BlockSpec((1, TOKEN_BLOCK), lambda i: (0, jnp.maximum(i - nb, 0))),
            pl.BlockSpec((TOKEN_BLOCK, d), lambda i: (i, 0)),
            pl.BlockSpec((K, d), lambda i: (0, 0)),
        ],
        out_specs=pl.BlockSpec((TOKEN_BLOCK, d), lambda i: (i, 0)),
        out_shape=jax.ShapeDtypeStruct((bn, d), jnp.float32),
        compiler_params=pltpu.CompilerParams(
            dimension_semantics=("arbitrary",),
        ),
    )(lt1, lt2, thr1.reshape(1, cn), thr2.reshape(1, cn), tf, E)
    return out.reshape(b, n, d)


# speedup vs baseline: 1.0093x; 1.0093x over previous
import functools

import jax
import jax.numpy as jnp
from jax import lax
from jax.experimental import pallas as pl
from jax.experimental.pallas import tpu as pltpu
from jax.experimental.pallas import tpu_sc as plsc

TAU = 10.0
TOPK = 8
K = 64
D = 2048
TOKEN_BLOCK = 1024

_SC_INFO = plsc.get_sparse_core_info()
_NWORKERS = _SC_INFO.num_cores * _SC_INFO.num_subcores
_LANES = _SC_INFO.num_lanes


def _logits_block(z_ref, cb_ref, lt_ref):
    z = z_ref[...]
    cb = cb_ref[...]
    raw = lax.dot_general(
        cb, z, (((1,), (1,)), ((), ())), preferred_element_type=jnp.float32)
    z2 = lax.dot_general(
        jnp.ones((1, z.shape[1]), jnp.float32), z * z,
        (((1,), (1,)), ((), ())), preferred_element_type=jnp.float32)
    rz = TAU / jnp.maximum(jnp.sqrt(z2), 1e-12)
    c2 = lax.dot_general(
        cb * cb, jnp.ones((z.shape[1], 8), jnp.float32),
        (((1,), (0,)), ((), ())), preferred_element_type=jnp.float32)[:, :1]
    rc = 1.0 / jnp.maximum(jnp.sqrt(c2), 1e-12)
    lt_ref[...] = raw * rz * rc


def _route_sc(bn):
    tpw = bn // _NWORKERS
    groups = tpw // _LANES
    mesh = plsc.VectorSubcoreMesh(core_axis_name="c", subcore_axis_name="s")

    @functools.partial(
        pl.kernel,
        mesh=mesh,
        out_type=jax.ShapeDtypeStruct((bn,), jnp.float32),
        scratch_types=[
            pltpu.VMEM((K, tpw), jnp.float32),
            pltpu.VMEM((tpw,), jnp.float32),
        ],
    )
    def route(lt_hbm, thr_hbm, lv, tv):
        wid = lax.axis_index("s") * _SC_INFO.num_cores + lax.axis_index("c")
        base = wid * tpw
        pltpu.sync_copy(lt_hbm.at[:, pl.ds(base, tpw)], lv)

        def body(g, carry):
            neg = jnp.full((_LANES,), -jnp.inf, jnp.float32)
            s = [neg] * TOPK
            col = g * _LANES
            for k in range(K):
                x = lv[k, pl.ds(col, _LANES)]
                for i in range(TOPK):
                    hi = jnp.maximum(s[i], x)
                    x = jnp.minimum(s[i], x)
                    s[i] = hi
            tv[pl.ds(col, _LANES)] = s[TOPK - 1]
            return carry

        lax.fori_loop(0, groups, body, 0)
        pltpu.sync_copy(tv, thr_hbm.at[pl.ds(base, tpw)])

    return route


def _combine_block(lt_ref, thr_ref, t_ref, e_ref, o_ref):
    lt = lt_ref[...]
    thr = thr_ref[...]
    e = jnp.where(lt >= thr, jnp.exp(lt - thr), 0.0)
    s = jnp.sum(e, axis=0, keepdims=True)
    w = e * (1.0 / s)
    g = lax.dot_general(
        w, e_ref[...], (((0,), (0,)), ((), ())), preferred_element_type=jnp.float32)
    o_ref[...] = t_ref[...] * (1.0 + g)


@functools.partial(jax.jit, static_argnames=())
def kernel(z, target, codebook, E):
    b, n, d = z.shape
    bn = b * n
    nb = bn // TOKEN_BLOCK
    zf = z.reshape(bn, d)
    tf = target.reshape(bn, d)

    lt = pl.pallas_call(
        _logits_block,
        grid=(nb,),
        in_specs=[
            pl.BlockSpec((TOKEN_BLOCK, d), lambda i: (i, 0)),
            pl.BlockSpec((K, d), lambda i: (0, 0)),
        ],
        out_specs=pl.BlockSpec((K, TOKEN_BLOCK), lambda i: (0, i)),
        out_shape=jax.ShapeDtypeStruct((K, bn), jnp.float32),
        compiler_params=pltpu.CompilerParams(
            dimension_semantics=("arbitrary",),
        ),
    )(zf, codebook)

    thr = _route_sc(bn)(lt)

    out = pl.pallas_call(
        _combine_block,
        grid=(nb,),
        in_specs=[
            pl.BlockSpec((K, TOKEN_BLOCK), lambda i: (0, i)),
            pl.BlockSpec((1, TOKEN_BLOCK), lambda i: (0, i)),
            pl.BlockSpec((TOKEN_BLOCK, d), lambda i: (i, 0)),
            pl.BlockSpec((K, d), lambda i: (0, 0)),
        ],
        out_specs=pl.BlockSpec((TOKEN_BLOCK, d), lambda i: (i, 0)),
        out_shape=jax.ShapeDtypeStruct((bn, d), jnp.float32),
        compiler_params=pltpu.CompilerParams(
            dimension_semantics=("arbitrary",),
        ),
    )(lt, thr.reshape(1, bn), tf, E)
    return out.reshape(b, n, d)


# speedup vs baseline: 1.0342x; 1.0247x over previous
import functools

import jax
import jax.numpy as jnp
from jax import lax
from jax.experimental import pallas as pl
from jax.experimental.pallas import tpu as pltpu
from jax.experimental.pallas import tpu_sc as plsc

TAU = 10.0
TOPK = 8
K = 64
D = 2048
TOKEN_BLOCK = 1024
LOGITS_BLOCK = 2048

_SC_INFO = plsc.get_sparse_core_info()
_NWORKERS = _SC_INFO.num_cores * _SC_INFO.num_subcores
_LANES = _SC_INFO.num_lanes


def _logits_block(z_ref, cb_ref, lt_ref):
    z = z_ref[...]
    cb = cb_ref[...]
    raw = lax.dot_general(
        cb, z, (((1,), (1,)), ((), ())), preferred_element_type=jnp.float32)
    z2 = lax.dot_general(
        jnp.ones((1, z.shape[1]), jnp.float32), z * z,
        (((1,), (1,)), ((), ())), preferred_element_type=jnp.float32)
    rz = TAU / jnp.maximum(jnp.sqrt(z2), 1e-12)
    c2 = lax.dot_general(
        cb * cb, jnp.ones((z.shape[1], 8), jnp.float32),
        (((1,), (0,)), ((), ())), preferred_element_type=jnp.float32)[:, :1]
    rc = 1.0 / jnp.maximum(jnp.sqrt(c2), 1e-12)
    lt_ref[...] = raw * rz * rc


def _route_sc(bn):
    tpw = bn // _NWORKERS
    groups = tpw // _LANES
    mesh = plsc.VectorSubcoreMesh(core_axis_name="c", subcore_axis_name="s")

    @functools.partial(
        pl.kernel,
        mesh=mesh,
        out_type=jax.ShapeDtypeStruct((bn,), jnp.float32),
        scratch_types=[
            pltpu.VMEM((K, tpw), jnp.float32),
            pltpu.VMEM((tpw,), jnp.float32),
        ],
    )
    def route(lt_hbm, thr_hbm, lv, tv):
        wid = lax.axis_index("s") * _SC_INFO.num_cores + lax.axis_index("c")
        base = wid * tpw
        pltpu.sync_copy(lt_hbm.at[:, pl.ds(base, tpw)], lv)

        def body(g, carry):
            neg = jnp.full((_LANES,), -jnp.inf, jnp.float32)
            s = [neg] * TOPK
            col = g * _LANES
            for k in range(K):
                x = lv[k, pl.ds(col, _LANES)]
                for i in range(TOPK):
                    hi = jnp.maximum(s[i], x)
                    x = jnp.minimum(s[i], x)
                    s[i] = hi
            tv[pl.ds(col, _LANES)] = s[TOPK - 1]
            return carry

        lax.fori_loop(0, groups, body, 0)
        pltpu.sync_copy(tv, thr_hbm.at[pl.ds(base, tpw)])

    return route


def _combine_block(lt_ref, thr_ref, t_ref, e_ref, o_ref):
    lt = lt_ref[...]
    thr = thr_ref[...]
    e = jnp.where(lt >= thr, jnp.exp(lt - thr), 0.0)
    s = jnp.sum(e, axis=0, keepdims=True)
    w = e * (1.0 / s)
    g = lax.dot_general(
        w, e_ref[...], (((0,), (0,)), ((), ())), preferred_element_type=jnp.float32)
    o_ref[...] = t_ref[...] * (1.0 + g)


@functools.partial(jax.jit, static_argnames=())
def kernel(z, target, codebook, E):
    b, n, d = z.shape
    bn = b * n
    nb = bn // TOKEN_BLOCK
    zf = z.reshape(bn, d)
    tf = target.reshape(bn, d)

    lt = pl.pallas_call(
        _logits_block,
        grid=(bn // LOGITS_BLOCK,),
        in_specs=[
            pl.BlockSpec((LOGITS_BLOCK, d), lambda i: (i, 0)),
            pl.BlockSpec((K, d), lambda i: (0, 0)),
        ],
        out_specs=pl.BlockSpec((K, LOGITS_BLOCK), lambda i: (0, i)),
        out_shape=jax.ShapeDtypeStruct((K, bn), jnp.float32),
        compiler_params=pltpu.CompilerParams(
            dimension_semantics=("arbitrary",),
        ),
    )(zf, codebook)

    thr = _route_sc(bn)(lt)

    out = pl.pallas_call(
        _combine_block,
        grid=(nb,),
        in_specs=[
            pl.BlockSpec((K, TOKEN_BLOCK), lambda i: (0, i)),
            pl.BlockSpec((1, TOKEN_BLOCK), lambda i: (0, i)),
            pl.BlockSpec((TOKEN_BLOCK, d), lambda i: (i, 0)),
            pl.BlockSpec((K, d), lambda i: (0, 0)),
        ],
        out_specs=pl.BlockSpec((TOKEN_BLOCK, d), lambda i: (i, 0)),
        out_shape=jax.ShapeDtypeStruct((bn, d), jnp.float32),
        compiler_params=pltpu.CompilerParams(
            dimension_semantics=("arbitrary",),
        ),
    )(lt, thr.reshape(1, bn), tf, E)
    return out.reshape(b, n, d)
